# Initial kernel scaffold; baseline (speedup 1.0000x reference)
import jax
import jax.numpy as jnp
from jax.experimental import pallas as pl


def kernel(x, edge_index, edge_type, query_emb, x_batch, edge_batch, W):
    raise NotImplementedError("write your pallas kernel here")


# speedup vs baseline: 1.4527x; 1.4527x over previous
import functools

import jax
import jax.numpy as jnp
from jax import lax
from jax.experimental import pallas as pl
from jax.experimental.pallas import tpu as pltpu
from jax.experimental.pallas import tpu_sc as plsc

N_ROWS = 10000
D = 128
NC = 2
NS = 16
NW = NC * NS
PER = N_ROWS // NW
CH = 104
NCH = PER // CH
TAIL = N_ROWS - NW * PER
TAIL_BASE = NW * PER


def _gather_body(w_hbm, idx_hbm, out_hbm, idx_v, rows_v, idx_t, rows_t, sem):
    wid = lax.axis_index("s") * NC + lax.axis_index("c")
    base = wid * PER
    pltpu.sync_copy(idx_hbm.at[pl.ds(base, PER)], idx_v)
    descs = []
    for c in range(NCH):
        descs.append(pltpu.async_copy(
            w_hbm.at[idx_v.at[pl.ds(c * CH, CH)]],
            rows_v.at[pl.ds(c * CH, CH)], sem))
    for d in descs:
        d.wait()
    pltpu.sync_copy(rows_v, out_hbm.at[pl.ds(base, PER)])

    @pl.when(wid == 0)
    def _():
        pltpu.sync_copy(idx_hbm.at[pl.ds(TAIL_BASE, TAIL)], idx_t)
        pltpu.async_copy(w_hbm.at[idx_t], rows_t, sem).wait()
        pltpu.sync_copy(rows_t, out_hbm.at[pl.ds(TAIL_BASE, TAIL)])


_gather = functools.partial(
    pl.kernel,
    mesh=plsc.VectorSubcoreMesh(core_axis_name="c", subcore_axis_name="s"),
    out_type=jax.ShapeDtypeStruct((N_ROWS, D), jnp.float32),
    scratch_types=[
        pltpu.VMEM((PER,), jnp.int32),
        pltpu.VMEM((PER, D), jnp.float32),
        pltpu.VMEM((TAIL,), jnp.int32),
        pltpu.VMEM((TAIL, D), jnp.float32),
        pltpu.SemaphoreType.DMA,
    ],
)(_gather_body)


def kernel(x, edge_index, edge_type, query_emb, x_batch, edge_batch, W):
    return _gather(W, x.astype(jnp.int32))


# speedup vs baseline: 1.5005x; 1.0329x over previous
import functools

import jax
import jax.numpy as jnp
from jax import lax
from jax.experimental import pallas as pl
from jax.experimental.pallas import tpu as pltpu
from jax.experimental.pallas import tpu_sc as plsc

N_ROWS = 10000
D = 128
NC = 2
NS = 16
NW = NC * NS
PER = N_ROWS // NW
CH = 104
NCH = PER // CH
TAIL = N_ROWS - NW * PER
LAST = NW - 1


def _gather_body(w_hbm, idx_hbm, out_hbm, idx_v, rows_v,
                 g0, g1, g2, g3, wsem):
    wid = lax.axis_index("s") * NC + lax.axis_index("c")
    base = wid * PER
    pltpu.sync_copy(idx_hbm.at[pl.ds(base, PER)], idx_v.at[pl.ds(0, PER)])

    @pl.when(wid == LAST)
    def _():
        pltpu.sync_copy(idx_hbm.at[pl.ds(base + PER, TAIL)],
                        idx_v.at[pl.ds(PER, TAIL)])

    gsems = (g0, g1, g2)
    gd = []
    for c in range(NCH):
        gd.append(pltpu.async_copy(
            w_hbm.at[idx_v.at[pl.ds(c * CH, CH)]],
            rows_v.at[pl.ds(c * CH, CH)], gsems[c]))

    @pl.when(wid == LAST)
    def _():
        pltpu.async_copy(w_hbm.at[idx_v.at[pl.ds(PER, TAIL)]],
                         rows_v.at[pl.ds(PER, TAIL)], g3).wait()

    wd = []
    for c in range(NCH):
        gd[c].wait()
        wd.append(pltpu.async_copy(
            rows_v.at[pl.ds(c * CH, CH)],
            out_hbm.at[pl.ds(base + c * CH, CH)], wsem))

    @pl.when(wid == LAST)
    def _():
        pltpu.async_copy(rows_v.at[pl.ds(PER, TAIL)],
                         out_hbm.at[pl.ds(base + PER, TAIL)], wsem).wait()

    for d in wd:
        d.wait()


_gather = functools.partial(
    pl.kernel,
    mesh=plsc.VectorSubcoreMesh(core_axis_name="c", subcore_axis_name="s"),
    out_type=jax.ShapeDtypeStruct((N_ROWS, D), jnp.float32),
    scratch_types=[
        pltpu.VMEM((PER + TAIL,), jnp.int32),
        pltpu.VMEM((PER + TAIL, D), jnp.float32),
        pltpu.SemaphoreType.DMA,
        pltpu.SemaphoreType.DMA,
        pltpu.SemaphoreType.DMA,
        pltpu.SemaphoreType.DMA,
        pltpu.SemaphoreType.DMA,
    ],
)(_gather_body)


def kernel(x, edge_index, edge_type, query_emb, x_batch, edge_batch, W):
    return _gather(W, x.astype(jnp.int32))


# speedup vs baseline: 1.5105x; 1.0067x over previous
import functools

import jax
import jax.numpy as jnp
from jax import lax
from jax.experimental import pallas as pl
from jax.experimental.pallas import tpu as pltpu
from jax.experimental.pallas import tpu_sc as plsc

N_ROWS = 10000
D = 128
NC = 2
NS = 16
NW = NC * NS
PER = N_ROWS // NW
EXT = PER + (N_ROWS - NW * PER)


def _gather_body(w_hbm, idx_hbm, out_hbm, idx_v, rows_v, gsem):
    wid = lax.axis_index("s") * NC + lax.axis_index("c")
    base = wid * PER
    pltpu.sync_copy(idx_hbm.at[pl.ds(base, EXT)], idx_v)
    pltpu.async_copy(w_hbm.at[idx_v], rows_v, gsem).wait()
    pltpu.sync_copy(rows_v, out_hbm.at[pl.ds(base, EXT)])


_gather = functools.partial(
    pl.kernel,
    mesh=plsc.VectorSubcoreMesh(core_axis_name="c", subcore_axis_name="s"),
    out_type=jax.ShapeDtypeStruct((N_ROWS, D), jnp.float32),
    scratch_types=[
        pltpu.VMEM((EXT,), jnp.int32),
        pltpu.VMEM((EXT, D), jnp.float32),
        pltpu.SemaphoreType.DMA,
    ],
)(_gather_body)


def kernel(x, edge_index, edge_type, query_emb, x_batch, edge_batch, W):
    return _gather(W, x.astype(jnp.int32))


# speedup vs baseline: 1.5174x; 1.0045x over previous
import functools

import jax
import jax.numpy as jnp
from jax import lax
from jax.experimental import pallas as pl
from jax.experimental.pallas import tpu as pltpu
from jax.experimental.pallas import tpu_sc as plsc

N_ROWS = 10000
D = 128
NC = 2
NS = 16
NW = NC * NS
PER = N_ROWS // NW
EXT = PER + (N_ROWS - NW * PER)


C0 = 168
C1 = EXT - C0


def _gather_body(w_hbm, idx_hbm, out_hbm, idx_v, rows_v, g0, g1, wsem):
    wid = lax.axis_index("s") * NC + lax.axis_index("c")
    base = wid * PER
    pltpu.sync_copy(idx_hbm.at[pl.ds(base, EXT)], idx_v)
    d0 = pltpu.async_copy(w_hbm.at[idx_v.at[pl.ds(0, C0)]],
                          rows_v.at[pl.ds(0, C0)], g0)
    d1 = pltpu.async_copy(w_hbm.at[idx_v.at[pl.ds(C0, C1)]],
                          rows_v.at[pl.ds(C0, C1)], g1)
    d0.wait()
    w0 = pltpu.async_copy(rows_v.at[pl.ds(0, C0)],
                          out_hbm.at[pl.ds(base, C0)], wsem)
    d1.wait()
    pltpu.async_copy(rows_v.at[pl.ds(C0, C1)],
                     out_hbm.at[pl.ds(base + C0, C1)], wsem).wait()
    w0.wait()


_gather = functools.partial(
    pl.kernel,
    mesh=plsc.VectorSubcoreMesh(core_axis_name="c", subcore_axis_name="s"),
    out_type=jax.ShapeDtypeStruct((N_ROWS, D), jnp.float32),
    scratch_types=[
        pltpu.VMEM((EXT,), jnp.int32),
        pltpu.VMEM((EXT, D), jnp.float32),
        pltpu.SemaphoreType.DMA,
        pltpu.SemaphoreType.DMA,
        pltpu.SemaphoreType.DMA,
    ],
)(_gather_body)


def kernel(x, edge_index, edge_type, query_emb, x_batch, edge_batch, W):
    return _gather(W, x.astype(jnp.int32))
